# baseline (speedup 1.0000x reference)
import jax
import jax.numpy as jnp
from jax import lax
from jax.experimental import pallas as pl
from jax.experimental.pallas import tpu as pltpu
from jax.experimental.pallas import tpu_sc as plsc

N = 10000
DH = 256
HF = 128
NC = 2
NS = 16
LANES = 128
ACC_ROWS = 10240
EPS = 1e-5


def _rup(a, b):
    return -(-a // b) * b


def _sc_mesh():
    return plsc.VectorSubcoreMesh(core_axis_name="c", subcore_axis_name="s",
                                  num_cores=NC, num_subcores=NS)


def _make_deg_kernel(n_rows):
    rows_per_w = n_rows // (NC * NS)
    n_chunks = rows_per_w // 8
    zrows = ACC_ROWS // NS // LANES

    def body(col_hbm, out_hbm, ones_v, stage_v, zrow_v, acc_sh):
        c = lax.axis_index("c")
        s = lax.axis_index("s")
        for j in range(8):
            ones_v[pl.ds(16 * j, 16)] = jnp.ones((16,), jnp.float32)
            zrow_v[pl.ds(16 * j, 16)] = jnp.zeros((16,), jnp.float32)

        def zloop(t, carry):
            pltpu.sync_copy(zrow_v, acc_sh.at[pl.ds(s * 640 + t * 128, 128)])
            return carry

        lax.fori_loop(0, zrows, zloop, None)
        plsc.subcore_barrier()

        base = (c * NS + s) * rows_per_w

        def eloop(k, carry):
            pltpu.sync_copy(col_hbm.at[pl.ds(base + k * 8, 8)], stage_v)
            for j in range(8):
                pltpu.sync_copy(ones_v, acc_sh.at[stage_v.at[j]], add=True)
            return carry

        lax.fori_loop(0, n_chunks, eloop, None)
        plsc.subcore_barrier()
        pltpu.sync_copy(acc_sh.at[pl.ds(s * 640, 640)],
                        out_hbm.at[pl.ds(c * ACC_ROWS + s * 640, 640)])

    return pl.kernel(
        body,
        out_type=jax.ShapeDtypeStruct((NC * ACC_ROWS,), jnp.float32),
        mesh=_sc_mesh(),
        compiler_params=pltpu.CompilerParams(needs_layout_passes=False),
        scratch_types=[
            pltpu.VMEM((LANES,), jnp.float32),
            pltpu.VMEM((8, LANES), jnp.int32),
            pltpu.VMEM((LANES,), jnp.float32),
            pltpu.VMEM_SHARED((ACC_ROWS,), jnp.float32),
        ],
    )


def _make_agg_kernel(n_rows):
    units = n_rows // NS
    n_chunks = units // 8
    stripe = 624

    def body(hs_hbm, row3d_hbm, col2d_hbm, out_hbm,
             r0g, c0g, r1g, c1g, r2g, c2g, bufA, bufB, zbuf, acc_sh,
             gsA, gsB, ssA, ssB, st0, st1, st2, zsem):
        c = lax.axis_index("c")
        s = lax.axis_index("s")
        rstg = [r0g, r1g, r2g]
        cstg = [c0g, c1g, c2g]
        stsem = [st0, st1, st2]
        bufs = [bufA, bufB]
        gsem = [gsA, gsB]
        ssem = [ssA, ssB]

        base = s * units
        pend_stage = {}
        pend_gather = {}
        pend_scatter = {}

        def stage_fire(k):
            i = k % 3
            r0 = base + 8 * k
            pend_stage[k] = (
                pltpu.async_copy(row3d_hbm.at[c, pl.ds(r0, 8)], rstg[i],
                                 stsem[i]),
                pltpu.async_copy(col2d_hbm.at[pl.ds(r0, 8)], cstg[i],
                                 stsem[i]),
            )

        def gather_fire(u):
            k, j = divmod(u, 8)
            b = u % 2
            pend_gather[u] = pltpu.async_copy(
                hs_hbm.at[rstg[k % 3].at[j]], bufs[b], gsem[b])

        stage_fire(0)
        stage_fire(1)
        for i in range(16):
            for j in range(8):
                zbuf[i, pl.ds(16 * j, 16)] = jnp.zeros((16,), jnp.float32)
        zcps = [pltpu.async_copy(zbuf, acc_sh.at[pl.ds(s * 640 + t * 16, 16)],
                                 zsem) for t in range(40)]
        for cp in pend_stage[0]:
            cp.wait()
        gather_fire(0)
        gather_fire(1)
        for cp in zcps:
            cp.wait()
        plsc.subcore_barrier()

        for k in range(n_chunks):
            if k + 2 < n_chunks:
                stage_fire(k + 2)
            for j in range(8):
                u = 8 * k + j
                if j == 6 and k + 1 < n_chunks:
                    for cp in pend_stage[k + 1]:
                        cp.wait()
                b = u % 2
                pend_gather[u].wait()
                pend_scatter[u] = pltpu.async_copy(
                    bufs[b], acc_sh.at[cstg[k % 3].at[j]], ssem[b], add=True)
                pend_scatter[u].wait()
                if u + 2 < units:
                    gather_fire(u + 2)

        plsc.subcore_barrier()
        pltpu.sync_copy(acc_sh.at[pl.ds(s * stripe, stripe)],
                        out_hbm.at[pl.ds(c * N + s * stripe, stripe)])

        @pl.when(s == NS - 1)
        def _tail():
            pltpu.sync_copy(acc_sh.at[pl.ds(NS * stripe, N - NS * stripe)],
                            out_hbm.at[pl.ds(c * N + NS * stripe,
                                             N - NS * stripe)])

    return pl.kernel(
        body,
        out_type=jax.ShapeDtypeStruct((NC * N, HF), jnp.float32),
        mesh=_sc_mesh(),
        compiler_params=pltpu.CompilerParams(needs_layout_passes=False),
        scratch_types=[
            pltpu.VMEM((8, LANES), jnp.int32),
            pltpu.VMEM((8, LANES), jnp.int32),
            pltpu.VMEM((8, LANES), jnp.int32),
            pltpu.VMEM((8, LANES), jnp.int32),
            pltpu.VMEM((8, LANES), jnp.int32),
            pltpu.VMEM((8, LANES), jnp.int32),
            pltpu.VMEM((LANES, HF), jnp.float32),
            pltpu.VMEM((LANES, HF), jnp.float32),
            pltpu.VMEM((16, HF), jnp.float32),
            pltpu.VMEM_SHARED((ACC_ROWS, HF), jnp.float32),
            pltpu.SemaphoreType.DMA,
            pltpu.SemaphoreType.DMA,
            pltpu.SemaphoreType.DMA,
            pltpu.SemaphoreType.DMA,
            pltpu.SemaphoreType.DMA,
            pltpu.SemaphoreType.DMA,
            pltpu.SemaphoreType.DMA,
            pltpu.SemaphoreType.DMA,
        ],
    )


def _make_edgeout_kernel(n_rows):
    rows_per_w = n_rows // (NC * NS)
    n_chunks = rows_per_w // 8
    out_per_w = rows_per_w * LANES * 2

    def body(hab_hbm, src_hbm, dst_hbm, out_hbm, tab, sstage, dstage, outbuf):
        c = lax.axis_index("c")
        s = lax.axis_index("s")
        w = c * NS + s
        pltpu.sync_copy(hab_hbm, tab)
        iota = lax.iota(jnp.int32, 16)
        base_row = w * rows_per_w

        def eloop(k, carry):
            pltpu.sync_copy(src_hbm.at[pl.ds(base_row + k * 8, 8)], sstage)
            pltpu.sync_copy(dst_hbm.at[pl.ds(base_row + k * 8, 8)], dstage)
            for j in range(8):
                for jj in range(8):
                    sl = pl.ds(jj * 16, 16)
                    src16 = sstage[j, sl]
                    dst16 = dstage[j, sl]
                    a0 = plsc.load_gather(tab, [src16 * 4])
                    a1 = plsc.load_gather(tab, [src16 * 4 + 1])
                    b0 = plsc.load_gather(tab, [dst16 * 4 + 2])
                    b1 = plsc.load_gather(tab, [dst16 * 4 + 3])
                    p = ((k * 8 + j) * LANES + jj * 16 + iota) * 2
                    plsc.store_scatter(outbuf, [p], a0 + b0)
                    plsc.store_scatter(outbuf, [p + 1], a1 + b1)
            return carry

        lax.fori_loop(0, n_chunks, eloop, None)
        pltpu.sync_copy(outbuf, out_hbm.at[pl.ds(w * out_per_w, out_per_w)])

    return pl.kernel(
        body,
        out_type=jax.ShapeDtypeStruct((n_rows * LANES * 2,), jnp.float32),
        mesh=_sc_mesh(),
        compiler_params=pltpu.CompilerParams(needs_layout_passes=False),
        scratch_types=[
            pltpu.VMEM((4 * N,), jnp.float32),
            pltpu.VMEM((8, LANES), jnp.int32),
            pltpu.VMEM((8, LANES), jnp.int32),
            pltpu.VMEM((out_per_w,), jnp.float32),
        ],
    )


def _tc_prologue(x_ref, we_ref, be_ref, w0_ref, degp_ref,
                 h_ref, hs_ref, dinv_ref):
    deg = degp_ref[0, :N, :] + degp_ref[1, :N, :]
    dinv = lax.rsqrt(deg)
    dinv_ref[...] = dinv
    h = jnp.dot(x_ref[...], we_ref[...],
                preferred_element_type=jnp.float32) + be_ref[...]
    h_ref[0, :, :] = h[:, :HF]
    h_ref[1, :, :] = h[:, HF:]
    hw = jnp.dot(h, w0_ref[...], preferred_element_type=jnp.float32)
    hs = hw * dinv
    hs_ref[0, :, :] = hs[:, :HF]
    hs_ref[1, :, :] = hs[:, HF:]


def _layer_core(h_ref, agg_ref, dinv_ref, b_ref, g_ref, bt_ref, hh):
    dinv = dinv_ref[...]
    cf = agg_ref[hh] * dinv + b_ref[hh]
    mean = jnp.mean(cf, axis=0, keepdims=True)
    d = cf - mean
    var = jnp.mean(d * d, axis=0, keepdims=True)
    cn = d * lax.rsqrt(var + EPS) * g_ref[hh] + bt_ref[hh]
    return h_ref[hh] + jnp.maximum(cn, 0.0)


def _tc_layer(h_ref, agg_ref, dinv_ref, b_ref, g_ref, bt_ref, wn_ref,
              hn_ref, hsn_ref):
    h2s = []
    for hh in range(2):
        h2 = _layer_core(h_ref, agg_ref, dinv_ref, b_ref, g_ref, bt_ref, hh)
        hn_ref[hh] = h2
        h2s.append(h2)
    hw = (jnp.dot(h2s[0], wn_ref[0], preferred_element_type=jnp.float32)
          + jnp.dot(h2s[1], wn_ref[1], preferred_element_type=jnp.float32))
    hs = hw * dinv_ref[...]
    hsn_ref[0, :, :] = hs[:, :HF]
    hsn_ref[1, :, :] = hs[:, HF:]


def _tc_layer_last(h_ref, agg_ref, dinv_ref, b_ref, g_ref, bt_ref,
                   w4_ref, b4_ref, hab_ref):
    h2s = [_layer_core(h_ref, agg_ref, dinv_ref, b_ref, g_ref, bt_ref, hh)
           for hh in range(2)]
    hab_ref[...] = (jnp.dot(h2s[0], w4_ref[0], preferred_element_type=jnp.float32)
                    + jnp.dot(h2s[1], w4_ref[1],
                              preferred_element_type=jnp.float32)
                    + b4_ref[...])


def kernel(x_list, edge_index_list, edge_index_out_list, W_embed, b_embed,
           W_conv, b_conv, gamma, beta, W_fc, b_fc):
    f32 = jnp.float32
    ei = edge_index_list.astype(jnp.int32)
    eo = edge_index_out_list.astype(jnp.int32)
    E = ei.shape[1]
    L = W_conv.shape[0]
    loop = jnp.arange(N, dtype=jnp.int32)

    ew = E + N
    rows = _rup(-(-ew // LANES), 8 * NS)
    padn = rows * LANES - ew
    fill = jnp.arange(padn, dtype=jnp.int32)
    rowp = jnp.concatenate([ei[0], loop, fill % N])
    colp = jnp.concatenate([ei[1], loop, N + fill % (ACC_ROWS - N)])
    row3d = jnp.stack([rowp, rowp + N]).reshape(2, rows, LANES)
    col2d = colp.reshape(rows, LANES)

    drows = _rup(-(-ew // LANES), 8 * NC * NS)
    dpad = drows * LANES - ew
    dfill = jnp.arange(dpad, dtype=jnp.int32)
    dcol2d = jnp.concatenate(
        [ei[1], loop, N + dfill % (ACC_ROWS - N)]).reshape(drows, LANES)

    orows = _rup(-(-E // LANES), 8 * NC * NS)
    opad = orows * LANES - E
    ofill = jnp.arange(opad, dtype=jnp.int32) % N
    srcp = jnp.concatenate([eo[0], ofill]).reshape(orows, LANES)
    dstp = jnp.concatenate([eo[1], ofill]).reshape(orows, LANES)

    deg_partials = _make_deg_kernel(drows)(dcol2d)
    degp = deg_partials.reshape(2, ACC_ROWS, 1)

    h, hs, dinv = pl.pallas_call(
        _tc_prologue,
        out_shape=(
            jax.ShapeDtypeStruct((2, N, HF), f32),
            jax.ShapeDtypeStruct((2, N, HF), f32),
            jax.ShapeDtypeStruct((N, 1), f32),
        ),
    )(x_list, W_embed, b_embed.reshape(1, DH), W_conv[0], degp)

    agg_fn = _make_agg_kernel(rows)
    for i in range(L):
        agg = agg_fn(hs.reshape(NC * N, HF), row3d, col2d).reshape(2, N, HF)
        b_i = b_conv[i].reshape(2, 1, HF)
        g_i = gamma[i].reshape(2, 1, HF)
        bt_i = beta[i].reshape(2, 1, HF)
        if i + 1 < L:
            h, hs = pl.pallas_call(
                _tc_layer,
                out_shape=(
                    jax.ShapeDtypeStruct((2, N, HF), f32),
                    jax.ShapeDtypeStruct((2, N, HF), f32),
                ),
            )(h, agg, dinv, b_i, g_i, bt_i, W_conv[i + 1].reshape(2, HF, DH))
        else:
            w4 = jnp.concatenate([W_fc[:DH], W_fc[DH:]], axis=1).reshape(2, HF, 4)
            b4 = jnp.concatenate([b_fc, jnp.zeros((2,), f32)]).reshape(1, 4)
            hab = pl.pallas_call(
                _tc_layer_last,
                out_shape=jax.ShapeDtypeStruct((N, 4), f32),
            )(h, agg, dinv, b_i, g_i, bt_i, w4, b4)

    eout = _make_edgeout_kernel(orows)(hab.reshape(4 * N), srcp, dstp)
    return eout.reshape(orows * LANES, 2)[:E]


# speedup vs baseline: 1.0147x; 1.0147x over previous
import jax
import jax.numpy as jnp
from jax import lax
from jax.experimental import pallas as pl
from jax.experimental.pallas import tpu as pltpu
from jax.experimental.pallas import tpu_sc as plsc

N = 10000
DH = 256
HF = 128
NC = 2
NS = 16
LANES = 128
ACC_ROWS = 10240
EPS = 1e-5


def _rup(a, b):
    return -(-a // b) * b


def _sc_mesh():
    return plsc.VectorSubcoreMesh(core_axis_name="c", subcore_axis_name="s",
                                  num_cores=NC, num_subcores=NS)


def _make_deg_kernel(n_rows):
    rows_per_w = n_rows // (NC * NS)
    n_chunks = rows_per_w // 8
    zrows = ACC_ROWS // NS // LANES

    def body(col_hbm, out_hbm, ones_v, stage_v, zrow_v, acc_sh):
        c = lax.axis_index("c")
        s = lax.axis_index("s")
        for j in range(8):
            ones_v[pl.ds(16 * j, 16)] = jnp.ones((16,), jnp.float32)
            zrow_v[pl.ds(16 * j, 16)] = jnp.zeros((16,), jnp.float32)

        def zloop(t, carry):
            pltpu.sync_copy(zrow_v, acc_sh.at[pl.ds(s * 640 + t * 128, 128)])
            return carry

        lax.fori_loop(0, zrows, zloop, None)
        plsc.subcore_barrier()

        base = (c * NS + s) * rows_per_w

        def eloop(k, carry):
            pltpu.sync_copy(col_hbm.at[pl.ds(base + k * 8, 8)], stage_v)
            for j in range(8):
                pltpu.sync_copy(ones_v, acc_sh.at[stage_v.at[j]], add=True)
            return carry

        lax.fori_loop(0, n_chunks, eloop, None)
        plsc.subcore_barrier()
        pltpu.sync_copy(acc_sh.at[pl.ds(s * 640, 640)],
                        out_hbm.at[pl.ds(c * ACC_ROWS + s * 640, 640)])

    return pl.kernel(
        body,
        out_type=jax.ShapeDtypeStruct((NC * ACC_ROWS,), jnp.float32),
        mesh=_sc_mesh(),
        compiler_params=pltpu.CompilerParams(needs_layout_passes=False),
        scratch_types=[
            pltpu.VMEM((LANES,), jnp.float32),
            pltpu.VMEM((8, LANES), jnp.int32),
            pltpu.VMEM((LANES,), jnp.float32),
            pltpu.VMEM_SHARED((ACC_ROWS,), jnp.float32),
        ],
    )


def _make_agg_kernel(n_rows):
    units = n_rows // NS
    n_chunks = units // 8
    stripe = 624

    def body(hs_hbm, row3d_hbm, col2d_hbm, out_hbm,
             r0g, c0g, r1g, c1g, r2g, c2g, bufA, bufB, zbuf, acc_sh,
             gsA, gsB, ssA, ssB, st0, st1, st2, zsem):
        c = lax.axis_index("c")
        s = lax.axis_index("s")
        rstg = [r0g, r1g, r2g]
        cstg = [c0g, c1g, c2g]
        stsem = [st0, st1, st2]
        bufs = [bufA, bufB]
        gsem = [gsA, gsB]
        ssem = [ssA, ssB]

        base = s * units
        pend_stage = {}
        pend_gather = {}
        pend_scatter = {}

        def stage_fire(k):
            i = k % 3
            r0 = base + 8 * k
            pend_stage[k] = (
                pltpu.async_copy(row3d_hbm.at[c, pl.ds(r0, 8)], rstg[i],
                                 stsem[i]),
                pltpu.async_copy(col2d_hbm.at[pl.ds(r0, 8)], cstg[i],
                                 stsem[i]),
            )

        def gather_fire(u):
            k, j = divmod(u, 8)
            b = u % 2
            pend_gather[u] = pltpu.async_copy(
                hs_hbm.at[rstg[k % 3].at[j]], bufs[b], gsem[b])

        stage_fire(0)
        stage_fire(1)
        for i in range(16):
            for j in range(8):
                zbuf[i, pl.ds(16 * j, 16)] = jnp.zeros((16,), jnp.float32)
        zcps = [pltpu.async_copy(zbuf, acc_sh.at[pl.ds(s * 640 + t * 16, 16)],
                                 zsem) for t in range(40)]
        for cp in pend_stage[0]:
            cp.wait()
        gather_fire(0)
        gather_fire(1)
        for cp in zcps:
            cp.wait()
        plsc.subcore_barrier()

        for k in range(n_chunks):
            if k + 2 < n_chunks:
                stage_fire(k + 2)
            for j in range(8):
                u = 8 * k + j
                if j == 6 and k + 1 < n_chunks:
                    for cp in pend_stage[k + 1]:
                        cp.wait()
                b = u % 2
                pend_gather[u].wait()
                pend_scatter[u] = pltpu.async_copy(
                    bufs[b], acc_sh.at[cstg[k % 3].at[j]], ssem[b], add=True)
                pend_scatter[u].wait()
                if u + 2 < units:
                    gather_fire(u + 2)

        plsc.subcore_barrier()
        pltpu.sync_copy(acc_sh.at[pl.ds(s * stripe, stripe)],
                        out_hbm.at[pl.ds(c * N + s * stripe, stripe)])

        @pl.when(s == NS - 1)
        def _tail():
            pltpu.sync_copy(acc_sh.at[pl.ds(NS * stripe, N - NS * stripe)],
                            out_hbm.at[pl.ds(c * N + NS * stripe,
                                             N - NS * stripe)])

    return pl.kernel(
        body,
        out_type=jax.ShapeDtypeStruct((NC * N, HF), jnp.float32),
        mesh=_sc_mesh(),
        compiler_params=pltpu.CompilerParams(needs_layout_passes=False),
        scratch_types=[
            pltpu.VMEM((8, LANES), jnp.int32),
            pltpu.VMEM((8, LANES), jnp.int32),
            pltpu.VMEM((8, LANES), jnp.int32),
            pltpu.VMEM((8, LANES), jnp.int32),
            pltpu.VMEM((8, LANES), jnp.int32),
            pltpu.VMEM((8, LANES), jnp.int32),
            pltpu.VMEM((LANES, HF), jnp.float32),
            pltpu.VMEM((LANES, HF), jnp.float32),
            pltpu.VMEM((16, HF), jnp.float32),
            pltpu.VMEM_SHARED((ACC_ROWS, HF), jnp.float32),
            pltpu.SemaphoreType.DMA,
            pltpu.SemaphoreType.DMA,
            pltpu.SemaphoreType.DMA,
            pltpu.SemaphoreType.DMA,
            pltpu.SemaphoreType.DMA,
            pltpu.SemaphoreType.DMA,
            pltpu.SemaphoreType.DMA,
            pltpu.SemaphoreType.DMA,
        ],
    )


def _make_edgeout_kernel(n_rows):
    rows_per_w = n_rows // (NC * NS)
    n_chunks = rows_per_w // 8
    out_per_w = rows_per_w * LANES * 2

    def body(hab_hbm, src_hbm, dst_hbm, out_hbm, tab, sstage, dstage, outbuf):
        c = lax.axis_index("c")
        s = lax.axis_index("s")
        w = c * NS + s
        pltpu.sync_copy(hab_hbm, tab)
        iota = lax.iota(jnp.int32, 16)
        base_row = w * rows_per_w

        def eloop(k, carry):
            pltpu.sync_copy(src_hbm.at[pl.ds(base_row + k * 8, 8)], sstage)
            pltpu.sync_copy(dst_hbm.at[pl.ds(base_row + k * 8, 8)], dstage)
            for j in range(8):
                for jj in range(8):
                    sl = pl.ds(jj * 16, 16)
                    src16 = sstage[j, sl]
                    dst16 = dstage[j, sl]
                    a0 = plsc.load_gather(tab, [src16 * 4])
                    a1 = plsc.load_gather(tab, [src16 * 4 + 1])
                    b0 = plsc.load_gather(tab, [dst16 * 4 + 2])
                    b1 = plsc.load_gather(tab, [dst16 * 4 + 3])
                    p = ((k * 8 + j) * LANES + jj * 16 + iota) * 2
                    plsc.store_scatter(outbuf, [p], a0 + b0)
                    plsc.store_scatter(outbuf, [p + 1], a1 + b1)
            return carry

        lax.fori_loop(0, n_chunks, eloop, None)
        pltpu.sync_copy(outbuf, out_hbm.at[pl.ds(w * out_per_w, out_per_w)])

    return pl.kernel(
        body,
        out_type=jax.ShapeDtypeStruct((n_rows * LANES * 2,), jnp.float32),
        mesh=_sc_mesh(),
        compiler_params=pltpu.CompilerParams(needs_layout_passes=False),
        scratch_types=[
            pltpu.VMEM((4 * N,), jnp.float32),
            pltpu.VMEM((8, LANES), jnp.int32),
            pltpu.VMEM((8, LANES), jnp.int32),
            pltpu.VMEM((out_per_w,), jnp.float32),
        ],
    )


def _tc_prologue(x_ref, we_ref, be_ref, w0_ref, degp_ref,
                 h_ref, hs_ref, dinv_ref):
    deg = degp_ref[0, :N, :] + degp_ref[1, :N, :]
    dinv = lax.rsqrt(deg)
    dinv_ref[...] = jnp.broadcast_to(dinv, (N, HF))
    h = jnp.dot(x_ref[...], we_ref[...],
                preferred_element_type=jnp.float32) + be_ref[...]
    h_ref[0, :, :] = h[:, :HF]
    h_ref[1, :, :] = h[:, HF:]
    hw = jnp.dot(h, w0_ref[...], preferred_element_type=jnp.float32)
    hs_ref[0, :, :] = hw[:, :HF] * dinv
    hs_ref[1, :, :] = hw[:, HF:] * dinv


def _layer_core(h_ref, agg_ref, dinv_ref, b_ref, g_ref, bt_ref, hh):
    cf = agg_ref[hh] * dinv_ref[...] + b_ref[hh]
    mean = jnp.mean(cf, axis=0, keepdims=True)
    d = cf - mean
    var = jnp.mean(d * d, axis=0, keepdims=True)
    cn = d * lax.rsqrt(var + EPS) * g_ref[hh] + bt_ref[hh]
    return h_ref[hh] + jnp.maximum(cn, 0.0)


def _tc_layer(h_ref, agg_ref, dinv_ref, b_ref, g_ref, bt_ref, wn_ref,
              hn_ref, hsn_ref):
    h2s = []
    for hh in range(2):
        h2 = _layer_core(h_ref, agg_ref, dinv_ref, b_ref, g_ref, bt_ref, hh)
        hn_ref[hh] = h2
        h2s.append(h2)
    hw = (jnp.dot(h2s[0], wn_ref[0], preferred_element_type=jnp.float32)
          + jnp.dot(h2s[1], wn_ref[1], preferred_element_type=jnp.float32))
    dinv = dinv_ref[...]
    hsn_ref[0, :, :] = hw[:, :HF] * dinv
    hsn_ref[1, :, :] = hw[:, HF:] * dinv


def _tc_layer_last(h_ref, agg_ref, dinv_ref, b_ref, g_ref, bt_ref,
                   w4_ref, b4_ref, hab_ref):
    h2s = [_layer_core(h_ref, agg_ref, dinv_ref, b_ref, g_ref, bt_ref, hh)
           for hh in range(2)]
    hab_ref[...] = (jnp.dot(h2s[0], w4_ref[0], preferred_element_type=jnp.float32)
                    + jnp.dot(h2s[1], w4_ref[1],
                              preferred_element_type=jnp.float32)
                    + b4_ref[...])


def kernel(x_list, edge_index_list, edge_index_out_list, W_embed, b_embed,
           W_conv, b_conv, gamma, beta, W_fc, b_fc):
    f32 = jnp.float32
    ei = edge_index_list.astype(jnp.int32)
    eo = edge_index_out_list.astype(jnp.int32)
    E = ei.shape[1]
    L = W_conv.shape[0]
    loop = jnp.arange(N, dtype=jnp.int32)

    ew = E + N
    rows = _rup(-(-ew // LANES), 8 * NS)
    padn = rows * LANES - ew
    fill = jnp.arange(padn, dtype=jnp.int32)
    rowp = jnp.concatenate([ei[0], loop, fill % N])
    colp = jnp.concatenate([ei[1], loop, N + fill % (ACC_ROWS - N)])
    row3d = jnp.stack([rowp, rowp + N]).reshape(2, rows, LANES)
    col2d = colp.reshape(rows, LANES)

    drows = _rup(-(-ew // LANES), 8 * NC * NS)
    dpad = drows * LANES - ew
    dfill = jnp.arange(dpad, dtype=jnp.int32)
    dcol2d = jnp.concatenate(
        [ei[1], loop, N + dfill % (ACC_ROWS - N)]).reshape(drows, LANES)

    orows = _rup(-(-E // LANES), 8 * NC * NS)
    opad = orows * LANES - E
    ofill = jnp.arange(opad, dtype=jnp.int32) % N
    srcp = jnp.concatenate([eo[0], ofill]).reshape(orows, LANES)
    dstp = jnp.concatenate([eo[1], ofill]).reshape(orows, LANES)

    deg_partials = _make_deg_kernel(drows)(dcol2d)
    degp = deg_partials.reshape(2, ACC_ROWS, 1)

    h, hs, dinv = pl.pallas_call(
        _tc_prologue,
        out_shape=(
            jax.ShapeDtypeStruct((2, N, HF), f32),
            jax.ShapeDtypeStruct((2, N, HF), f32),
            jax.ShapeDtypeStruct((N, HF), f32),
        ),
    )(x_list, W_embed, b_embed.reshape(1, DH), W_conv[0], degp)

    agg_fn = _make_agg_kernel(rows)
    for i in range(L):
        agg = agg_fn(hs.reshape(NC * N, HF), row3d, col2d).reshape(2, N, HF)
        b_i = b_conv[i].reshape(2, 1, HF)
        g_i = gamma[i].reshape(2, 1, HF)
        bt_i = beta[i].reshape(2, 1, HF)
        if i + 1 < L:
            h, hs = pl.pallas_call(
                _tc_layer,
                out_shape=(
                    jax.ShapeDtypeStruct((2, N, HF), f32),
                    jax.ShapeDtypeStruct((2, N, HF), f32),
                ),
            )(h, agg, dinv, b_i, g_i, bt_i, W_conv[i + 1].reshape(2, HF, DH))
        else:
            w4 = jnp.concatenate([W_fc[:DH], W_fc[DH:]], axis=1).reshape(2, HF, 4)
            b4 = jnp.concatenate([b_fc, jnp.zeros((2,), f32)]).reshape(1, 4)
            hab = pl.pallas_call(
                _tc_layer_last,
                out_shape=jax.ShapeDtypeStruct((N, 4), f32),
            )(h, agg, dinv, b_i, g_i, bt_i, w4, b4)

    eout = _make_edgeout_kernel(orows)(hab.reshape(4 * N), srcp, dstp)
    return eout.reshape(orows * LANES, 2)[:E]


# speedup vs baseline: 1.0515x; 1.0363x over previous
import jax
import jax.numpy as jnp
from jax import lax
from jax.experimental import pallas as pl
from jax.experimental.pallas import tpu as pltpu
from jax.experimental.pallas import tpu_sc as plsc

N = 10000
DH = 256
HF = 128
NC = 2
NS = 16
LANES = 128
ACC_ROWS = 10240
EPS = 1e-5


def _rup(a, b):
    return -(-a // b) * b


def _sc_mesh():
    return plsc.VectorSubcoreMesh(core_axis_name="c", subcore_axis_name="s",
                                  num_cores=NC, num_subcores=NS)


def _make_deg_kernel(n_rows):
    rows_per_w = n_rows // (NC * NS)
    n_chunks = rows_per_w // 8
    zrows = ACC_ROWS // NS // LANES

    def body(col_hbm, out_hbm, ones_v, stage_v, zrow_v, acc_sh):
        c = lax.axis_index("c")
        s = lax.axis_index("s")
        for j in range(8):
            ones_v[pl.ds(16 * j, 16)] = jnp.ones((16,), jnp.float32)
            zrow_v[pl.ds(16 * j, 16)] = jnp.zeros((16,), jnp.float32)

        def zloop(t, carry):
            pltpu.sync_copy(zrow_v, acc_sh.at[pl.ds(s * 640 + t * 128, 128)])
            return carry

        lax.fori_loop(0, zrows, zloop, None)
        plsc.subcore_barrier()

        base = (c * NS + s) * rows_per_w

        def eloop(k, carry):
            pltpu.sync_copy(col_hbm.at[pl.ds(base + k * 8, 8)], stage_v)
            for j in range(8):
                pltpu.sync_copy(ones_v, acc_sh.at[stage_v.at[j]], add=True)
            return carry

        lax.fori_loop(0, n_chunks, eloop, None)
        plsc.subcore_barrier()
        pltpu.sync_copy(acc_sh.at[pl.ds(s * 640, 640)],
                        out_hbm.at[pl.ds(c * ACC_ROWS + s * 640, 640)])

    return pl.kernel(
        body,
        out_type=jax.ShapeDtypeStruct((NC * ACC_ROWS,), jnp.float32),
        mesh=_sc_mesh(),
        compiler_params=pltpu.CompilerParams(needs_layout_passes=False),
        scratch_types=[
            pltpu.VMEM((LANES,), jnp.float32),
            pltpu.VMEM((8, LANES), jnp.int32),
            pltpu.VMEM((LANES,), jnp.float32),
            pltpu.VMEM_SHARED((ACC_ROWS,), jnp.float32),
        ],
    )


def _make_agg_kernel(n_rows):
    units = n_rows // NS
    n_chunks = units // 8
    stripe = 624

    def body(hs_hbm, row3d_hbm, col2d_hbm, out_hbm,
             r0g, c0g, r1g, c1g, r2g, c2g, bufA, bufB, acc_sh,
             gsA, gsB, ssA, ssB, st0, st1, st2, zsem):
        c = lax.axis_index("c")
        s = lax.axis_index("s")
        rstg = [r0g, r1g, r2g]
        cstg = [c0g, c1g, c2g]
        stsem = [st0, st1, st2]
        bufs = [bufA, bufB]
        gsem = [gsA, gsB]
        ssem = [ssA, ssB]

        base = s * units
        pend_stage = {}
        pend_gather = {}
        pend_scatter = {}

        def stage_fire(k):
            i = k % 3
            r0 = base + 8 * k
            pend_stage[k] = (
                pltpu.async_copy(row3d_hbm.at[c, pl.ds(r0, 8)], rstg[i],
                                 stsem[i]),
                pltpu.async_copy(col2d_hbm.at[pl.ds(r0, 8)], cstg[i],
                                 stsem[i]),
            )

        def gather_fire(u):
            k, j = divmod(u, 8)
            b = u % 2
            pend_gather[u] = pltpu.async_copy(
                hs_hbm.at[rstg[k % 3].at[j]], bufs[b], gsem[b])

        stage_fire(0)
        stage_fire(1)
        init_cp = pltpu.async_copy(hs_hbm.at[pl.ds(c * N + s * stripe, stripe)],
                                   acc_sh.at[pl.ds(s * stripe, stripe)], zsem)

        @pl.when(s == NS - 1)
        def _init_tail():
            pltpu.sync_copy(hs_hbm.at[pl.ds(c * N + NS * stripe,
                                            N - NS * stripe)],
                            acc_sh.at[pl.ds(NS * stripe, N - NS * stripe)])

        for cp in pend_stage[0]:
            cp.wait()
        gather_fire(0)
        gather_fire(1)
        init_cp.wait()
        plsc.subcore_barrier()

        for k in range(n_chunks):
            if k + 2 < n_chunks:
                stage_fire(k + 2)
            for j in range(8):
                u = 8 * k + j
                if j == 6 and k + 1 < n_chunks:
                    for cp in pend_stage[k + 1]:
                        cp.wait()
                b = u % 2
                pend_gather[u].wait()
                pend_scatter[u] = pltpu.async_copy(
                    bufs[b], acc_sh.at[cstg[k % 3].at[j]], ssem[b], add=True)
                pend_scatter[u].wait()
                if u + 2 < units:
                    gather_fire(u + 2)

        plsc.subcore_barrier()
        pltpu.sync_copy(acc_sh.at[pl.ds(s * stripe, stripe)],
                        out_hbm.at[pl.ds(c * N + s * stripe, stripe)])

        @pl.when(s == NS - 1)
        def _tail():
            pltpu.sync_copy(acc_sh.at[pl.ds(NS * stripe, N - NS * stripe)],
                            out_hbm.at[pl.ds(c * N + NS * stripe,
                                             N - NS * stripe)])

    return pl.kernel(
        body,
        out_type=jax.ShapeDtypeStruct((NC * N, HF), jnp.float32),
        mesh=_sc_mesh(),
        compiler_params=pltpu.CompilerParams(needs_layout_passes=False),
        scratch_types=[
            pltpu.VMEM((8, LANES), jnp.int32),
            pltpu.VMEM((8, LANES), jnp.int32),
            pltpu.VMEM((8, LANES), jnp.int32),
            pltpu.VMEM((8, LANES), jnp.int32),
            pltpu.VMEM((8, LANES), jnp.int32),
            pltpu.VMEM((8, LANES), jnp.int32),
            pltpu.VMEM((LANES, HF), jnp.float32),
            pltpu.VMEM((LANES, HF), jnp.float32),
            pltpu.VMEM_SHARED((ACC_ROWS, HF), jnp.float32),
            pltpu.SemaphoreType.DMA,
            pltpu.SemaphoreType.DMA,
            pltpu.SemaphoreType.DMA,
            pltpu.SemaphoreType.DMA,
            pltpu.SemaphoreType.DMA,
            pltpu.SemaphoreType.DMA,
            pltpu.SemaphoreType.DMA,
            pltpu.SemaphoreType.DMA,
        ],
    )


def _make_edgeout_kernel(n_rows):
    rows_per_w = n_rows // (NC * NS)
    n_chunks = rows_per_w // 8
    out_per_w = rows_per_w * LANES * 2

    def body(hab_hbm, src_hbm, dst_hbm, out_hbm, tab, sstage, dstage, outbuf):
        c = lax.axis_index("c")
        s = lax.axis_index("s")
        w = c * NS + s
        pltpu.sync_copy(hab_hbm, tab)
        iota = lax.iota(jnp.int32, 16)
        base_row = w * rows_per_w

        def eloop(k, carry):
            pltpu.sync_copy(src_hbm.at[pl.ds(base_row + k * 8, 8)], sstage)
            pltpu.sync_copy(dst_hbm.at[pl.ds(base_row + k * 8, 8)], dstage)
            for j in range(8):
                for jj in range(8):
                    sl = pl.ds(jj * 16, 16)
                    src16 = sstage[j, sl]
                    dst16 = dstage[j, sl]
                    a0 = plsc.load_gather(tab, [src16 * 4])
                    a1 = plsc.load_gather(tab, [src16 * 4 + 1])
                    b0 = plsc.load_gather(tab, [dst16 * 4 + 2])
                    b1 = plsc.load_gather(tab, [dst16 * 4 + 3])
                    p = ((k * 8 + j) * LANES + jj * 16 + iota) * 2
                    plsc.store_scatter(outbuf, [p], a0 + b0)
                    plsc.store_scatter(outbuf, [p + 1], a1 + b1)
            return carry

        lax.fori_loop(0, n_chunks, eloop, None)
        pltpu.sync_copy(outbuf, out_hbm.at[pl.ds(w * out_per_w, out_per_w)])

    return pl.kernel(
        body,
        out_type=jax.ShapeDtypeStruct((n_rows * LANES * 2,), jnp.float32),
        mesh=_sc_mesh(),
        compiler_params=pltpu.CompilerParams(needs_layout_passes=False),
        scratch_types=[
            pltpu.VMEM((4 * N,), jnp.float32),
            pltpu.VMEM((8, LANES), jnp.int32),
            pltpu.VMEM((8, LANES), jnp.int32),
            pltpu.VMEM((out_per_w,), jnp.float32),
        ],
    )


def _tc_prologue(x_ref, we_ref, be_ref, w0_ref, degp_ref,
                 h_ref, hs_ref, dinv_ref):
    deg = degp_ref[0, :N, :] + degp_ref[1, :N, :] + 1.0
    dinv = lax.rsqrt(deg)
    dinv_ref[...] = jnp.broadcast_to(dinv, (N, HF))
    h = jnp.dot(x_ref[...], we_ref[...],
                preferred_element_type=jnp.float32) + be_ref[...]
    h_ref[0, :, :] = h[:, :HF]
    h_ref[1, :, :] = h[:, HF:]
    hw = jnp.dot(h, w0_ref[...], preferred_element_type=jnp.float32)
    hs_ref[0, :, :] = hw[:, :HF] * dinv
    hs_ref[1, :, :] = hw[:, HF:] * dinv


def _layer_core(h_ref, agg_ref, dinv_ref, b_ref, g_ref, bt_ref, hh):
    cf = agg_ref[hh] * dinv_ref[...] + b_ref[hh]
    mean = jnp.mean(cf, axis=0, keepdims=True)
    d = cf - mean
    var = jnp.mean(d * d, axis=0, keepdims=True)
    cn = d * lax.rsqrt(var + EPS) * g_ref[hh] + bt_ref[hh]
    return h_ref[hh] + jnp.maximum(cn, 0.0)


def _tc_layer(h_ref, agg_ref, dinv_ref, b_ref, g_ref, bt_ref, wn_ref,
              hn_ref, hsn_ref):
    h2s = []
    for hh in range(2):
        h2 = _layer_core(h_ref, agg_ref, dinv_ref, b_ref, g_ref, bt_ref, hh)
        hn_ref[hh] = h2
        h2s.append(h2)
    hw = (jnp.dot(h2s[0], wn_ref[0], preferred_element_type=jnp.float32)
          + jnp.dot(h2s[1], wn_ref[1], preferred_element_type=jnp.float32))
    dinv = dinv_ref[...]
    hsn_ref[0, :, :] = hw[:, :HF] * dinv
    hsn_ref[1, :, :] = hw[:, HF:] * dinv


def _tc_layer_last(h_ref, agg_ref, dinv_ref, b_ref, g_ref, bt_ref,
                   w4_ref, b4_ref, hab_ref):
    h2s = [_layer_core(h_ref, agg_ref, dinv_ref, b_ref, g_ref, bt_ref, hh)
           for hh in range(2)]
    hab_ref[...] = (jnp.dot(h2s[0], w4_ref[0], preferred_element_type=jnp.float32)
                    + jnp.dot(h2s[1], w4_ref[1],
                              preferred_element_type=jnp.float32)
                    + b4_ref[...])


def kernel(x_list, edge_index_list, edge_index_out_list, W_embed, b_embed,
           W_conv, b_conv, gamma, beta, W_fc, b_fc):
    f32 = jnp.float32
    ei = edge_index_list.astype(jnp.int32)
    eo = edge_index_out_list.astype(jnp.int32)
    E = ei.shape[1]
    L = W_conv.shape[0]

    rows = _rup(-(-E // LANES), 8 * NC * NS)
    padn = rows * LANES - E
    fill = jnp.arange(padn, dtype=jnp.int32)
    rowp = jnp.concatenate([ei[0], fill % N])
    colp = jnp.concatenate([ei[1], N + fill % (ACC_ROWS - N)])
    row3d = jnp.stack([rowp, rowp + N]).reshape(2, rows, LANES)
    col2d = colp.reshape(rows, LANES)

    orows = _rup(-(-E // LANES), 8 * NC * NS)
    opad = orows * LANES - E
    ofill = jnp.arange(opad, dtype=jnp.int32) % N
    srcp = jnp.concatenate([eo[0], ofill]).reshape(orows, LANES)
    dstp = jnp.concatenate([eo[1], ofill]).reshape(orows, LANES)

    deg_partials = _make_deg_kernel(rows)(col2d)
    degp = deg_partials.reshape(2, ACC_ROWS, 1)

    h, hs, dinv = pl.pallas_call(
        _tc_prologue,
        out_shape=(
            jax.ShapeDtypeStruct((2, N, HF), f32),
            jax.ShapeDtypeStruct((2, N, HF), f32),
            jax.ShapeDtypeStruct((N, HF), f32),
        ),
    )(x_list, W_embed, b_embed.reshape(1, DH), W_conv[0], degp)

    agg_fn = _make_agg_kernel(rows)
    for i in range(L):
        agg = agg_fn(hs.reshape(NC * N, HF), row3d, col2d).reshape(2, N, HF)
        b_i = b_conv[i].reshape(2, 1, HF)
        g_i = gamma[i].reshape(2, 1, HF)
        bt_i = beta[i].reshape(2, 1, HF)
        if i + 1 < L:
            h, hs = pl.pallas_call(
                _tc_layer,
                out_shape=(
                    jax.ShapeDtypeStruct((2, N, HF), f32),
                    jax.ShapeDtypeStruct((2, N, HF), f32),
                ),
            )(h, agg, dinv, b_i, g_i, bt_i,
              W_conv[i + 1].reshape(2, HF, DH))
        else:
            w4 = jnp.concatenate([W_fc[:DH], W_fc[DH:]], axis=1).reshape(2, HF, 4)
            b4 = jnp.concatenate([b_fc, jnp.zeros((2,), f32)]).reshape(1, 4)
            hab = pl.pallas_call(
                _tc_layer_last,
                out_shape=jax.ShapeDtypeStruct((N, 4), f32),
            )(h, agg, dinv, b_i, g_i, bt_i, w4, b4)

    eout = _make_edgeout_kernel(orows)(hab.reshape(4 * N), srcp, dstp)
    return eout.reshape(orows * LANES, 2)[:E]


# speedup vs baseline: 1.1491x; 1.0928x over previous
import jax
import jax.numpy as jnp
from jax import lax
from jax.experimental import pallas as pl
from jax.experimental.pallas import tpu as pltpu
from jax.experimental.pallas import tpu_sc as plsc

N = 10000
DH = 256
HF = 128
NC = 2
NS = 16
LANES = 128
ACC_ROWS = 10240
EPS = 1e-5


def _rup(a, b):
    return -(-a // b) * b


def _sc_mesh():
    return plsc.VectorSubcoreMesh(core_axis_name="c", subcore_axis_name="s",
                                  num_cores=NC, num_subcores=NS)


def _make_deg_kernel(n_rows):
    rows_per_w = n_rows // (NC * NS)
    n_chunks = rows_per_w // 8
    zrows = ACC_ROWS // NS // LANES

    def body(col_hbm, out_hbm, ones_v, stage_v, zrow_v, acc_sh):
        c = lax.axis_index("c")
        s = lax.axis_index("s")
        for j in range(8):
            ones_v[pl.ds(16 * j, 16)] = jnp.ones((16,), jnp.float32)
            zrow_v[pl.ds(16 * j, 16)] = jnp.zeros((16,), jnp.float32)

        def zloop(t, carry):
            pltpu.sync_copy(zrow_v, acc_sh.at[pl.ds(s * 640 + t * 128, 128)])
            return carry

        lax.fori_loop(0, zrows, zloop, None)
        plsc.subcore_barrier()

        base = (c * NS + s) * rows_per_w

        def eloop(k, carry):
            pltpu.sync_copy(col_hbm.at[pl.ds(base + k * 8, 8)], stage_v)
            for j in range(8):
                pltpu.sync_copy(ones_v, acc_sh.at[stage_v.at[j]], add=True)
            return carry

        lax.fori_loop(0, n_chunks, eloop, None)
        plsc.subcore_barrier()
        pltpu.sync_copy(acc_sh.at[pl.ds(s * 640, 640)],
                        out_hbm.at[pl.ds(c * ACC_ROWS + s * 640, 640)])

    return pl.kernel(
        body,
        out_type=jax.ShapeDtypeStruct((NC * ACC_ROWS,), jnp.float32),
        mesh=_sc_mesh(),
        compiler_params=pltpu.CompilerParams(needs_layout_passes=False),
        scratch_types=[
            pltpu.VMEM((LANES,), jnp.float32),
            pltpu.VMEM((8, LANES), jnp.int32),
            pltpu.VMEM((LANES,), jnp.float32),
            pltpu.VMEM_SHARED((ACC_ROWS,), jnp.float32),
        ],
    )


def _make_agg_kernel(n_rows):
    units = n_rows // NS
    n_chunks = units // 8
    stripe = 624

    def body(hs_hbm, row3d_hbm, col2d_hbm, out_hbm,
             r0g, c0g, r1g, c1g, r2g, c2g, bufA, bufB, acc_sh,
             gsA, gsB, ssA, ssB, st0, st1, st2, zsem):
        c = lax.axis_index("c")
        s = lax.axis_index("s")
        rstg = [r0g, r1g, r2g]
        cstg = [c0g, c1g, c2g]
        stsem = [st0, st1, st2]
        bufs = [bufA, bufB]
        gsem = [gsA, gsB]
        ssem = [ssA, ssB]

        base = s * units
        pend_stage = {}
        pend_gather = {}
        pend_scatter = {}

        def stage_fire(k):
            i = k % 3
            r0 = base + 8 * k
            pend_stage[k] = (
                pltpu.async_copy(row3d_hbm.at[c, pl.ds(r0, 8)], rstg[i],
                                 stsem[i]),
                pltpu.async_copy(col2d_hbm.at[pl.ds(r0, 8)], cstg[i],
                                 stsem[i]),
            )

        def gather_fire(u):
            k, j = divmod(u, 8)
            b = u % 2
            pend_gather[u] = pltpu.async_copy(
                hs_hbm.at[rstg[k % 3].at[j]], bufs[b], gsem[b])

        stage_fire(0)
        stage_fire(1)
        init_cp = pltpu.async_copy(hs_hbm.at[pl.ds(c * N + s * stripe, stripe)],
                                   acc_sh.at[pl.ds(s * stripe, stripe)], zsem)

        @pl.when(s == NS - 1)
        def _init_tail():
            pltpu.sync_copy(hs_hbm.at[pl.ds(c * N + NS * stripe,
                                            N - NS * stripe)],
                            acc_sh.at[pl.ds(NS * stripe, N - NS * stripe)])

        for cp in pend_stage[0]:
            cp.wait()
        gather_fire(0)
        gather_fire(1)
        init_cp.wait()
        plsc.subcore_barrier()

        for k in range(n_chunks):
            if k + 2 < n_chunks:
                stage_fire(k + 2)
            for j in range(8):
                u = 8 * k + j
                if j == 6 and k + 1 < n_chunks:
                    for cp in pend_stage[k + 1]:
                        cp.wait()
                b = u % 2
                pend_gather[u].wait()
                pend_scatter[u] = pltpu.async_copy(
                    bufs[b], acc_sh.at[cstg[k % 3].at[j]], ssem[b], add=True)
                pend_scatter[u].wait()
                if u + 2 < units:
                    gather_fire(u + 2)

        plsc.subcore_barrier()
        pltpu.sync_copy(acc_sh.at[pl.ds(s * stripe, stripe)],
                        out_hbm.at[pl.ds(c * N + s * stripe, stripe)])

        @pl.when(s == NS - 1)
        def _tail():
            pltpu.sync_copy(acc_sh.at[pl.ds(NS * stripe, N - NS * stripe)],
                            out_hbm.at[pl.ds(c * N + NS * stripe,
                                             N - NS * stripe)])

    return pl.kernel(
        body,
        out_type=jax.ShapeDtypeStruct((NC * N, HF), jnp.float32),
        mesh=_sc_mesh(),
        compiler_params=pltpu.CompilerParams(needs_layout_passes=False),
        scratch_types=[
            pltpu.VMEM((8, LANES), jnp.int32),
            pltpu.VMEM((8, LANES), jnp.int32),
            pltpu.VMEM((8, LANES), jnp.int32),
            pltpu.VMEM((8, LANES), jnp.int32),
            pltpu.VMEM((8, LANES), jnp.int32),
            pltpu.VMEM((8, LANES), jnp.int32),
            pltpu.VMEM((LANES, HF), jnp.float32),
            pltpu.VMEM((LANES, HF), jnp.float32),
            pltpu.VMEM_SHARED((ACC_ROWS, HF), jnp.float32),
            pltpu.SemaphoreType.DMA,
            pltpu.SemaphoreType.DMA,
            pltpu.SemaphoreType.DMA,
            pltpu.SemaphoreType.DMA,
            pltpu.SemaphoreType.DMA,
            pltpu.SemaphoreType.DMA,
            pltpu.SemaphoreType.DMA,
            pltpu.SemaphoreType.DMA,
        ],
    )


def _make_edgeout_kernel(n_rows):
    rows_per_w = n_rows // (NC * NS)
    n_chunks = rows_per_w // 8
    out_per_w = rows_per_w * LANES * 2

    edges_per_w = rows_per_w * LANES

    def body(hab_hbm, src_hbm, dst_hbm, out_hbm, tab, sstage, dstage, outbuf):
        c = lax.axis_index("c")
        s = lax.axis_index("s")
        w = c * NS + s
        pltpu.sync_copy(hab_hbm, tab)
        iota = lax.iota(jnp.int32, 16)
        base_row = w * rows_per_w

        def eloop(k, carry):
            pltpu.sync_copy(src_hbm.at[pl.ds(base_row + k * 8, 8)], sstage)
            pltpu.sync_copy(dst_hbm.at[pl.ds(base_row + k * 8, 8)], dstage)
            for j in range(8):
                for jj in range(8):
                    sl = pl.ds(jj * 16, 16)
                    src16 = sstage[j, sl]
                    dst16 = dstage[j, sl]
                    a0 = plsc.load_gather(tab, [src16 * 4])
                    a1 = plsc.load_gather(tab, [src16 * 4 + 1])
                    b0 = plsc.load_gather(tab, [dst16 * 4 + 2])
                    b1 = plsc.load_gather(tab, [dst16 * 4 + 3])
                    p = (k * 8 + j) * LANES + jj * 16 + iota
                    plsc.store_scatter(outbuf, [p], a0 + b0)
                    plsc.store_scatter(outbuf, [p + edges_per_w], a1 + b1)
            return carry

        lax.fori_loop(0, n_chunks, eloop, None)
        pltpu.sync_copy(outbuf.at[pl.ds(0, edges_per_w)],
                        out_hbm.at[0, pl.ds(w * edges_per_w, edges_per_w)])
        pltpu.sync_copy(outbuf.at[pl.ds(edges_per_w, edges_per_w)],
                        out_hbm.at[1, pl.ds(w * edges_per_w, edges_per_w)])

    return pl.kernel(
        body,
        out_type=jax.ShapeDtypeStruct((2, n_rows * LANES), jnp.float32),
        mesh=_sc_mesh(),
        compiler_params=pltpu.CompilerParams(needs_layout_passes=False),
        scratch_types=[
            pltpu.VMEM((4 * N,), jnp.float32),
            pltpu.VMEM((8, LANES), jnp.int32),
            pltpu.VMEM((8, LANES), jnp.int32),
            pltpu.VMEM((out_per_w,), jnp.float32),
        ],
    )


def _tc_prologue(x_ref, we_ref, be_ref, w0_ref, degp_ref,
                 h_ref, hs_ref, dinv_ref):
    deg = degp_ref[0, :N, :] + degp_ref[1, :N, :] + 1.0
    dinv = lax.rsqrt(deg)
    dinv_ref[...] = jnp.broadcast_to(dinv, (N, HF))
    h = jnp.dot(x_ref[...], we_ref[...],
                preferred_element_type=jnp.float32) + be_ref[...]
    h_ref[0, :, :] = h[:, :HF]
    h_ref[1, :, :] = h[:, HF:]
    hw = jnp.dot(h, w0_ref[...], preferred_element_type=jnp.float32)
    hs_ref[0, :, :] = hw[:, :HF] * dinv
    hs_ref[1, :, :] = hw[:, HF:] * dinv


def _layer_core(h_ref, agg_ref, dinv_ref, b_ref, g_ref, bt_ref, hh):
    cf = agg_ref[hh] * dinv_ref[...] + b_ref[hh]
    mean = jnp.mean(cf, axis=0, keepdims=True)
    d = cf - mean
    var = jnp.mean(d * d, axis=0, keepdims=True)
    cn = d * lax.rsqrt(var + EPS) * g_ref[hh] + bt_ref[hh]
    return h_ref[hh] + jnp.maximum(cn, 0.0)


def _tc_layer(h_ref, agg_ref, dinv_ref, b_ref, g_ref, bt_ref, wn_ref,
              hn_ref, hsn_ref):
    h2s = []
    for hh in range(2):
        h2 = _layer_core(h_ref, agg_ref, dinv_ref, b_ref, g_ref, bt_ref, hh)
        hn_ref[hh] = h2
        h2s.append(h2)
    hw = (jnp.dot(h2s[0], wn_ref[0], preferred_element_type=jnp.float32)
          + jnp.dot(h2s[1], wn_ref[1], preferred_element_type=jnp.float32))
    dinv = dinv_ref[...]
    hsn_ref[0, :, :] = hw[:, :HF] * dinv
    hsn_ref[1, :, :] = hw[:, HF:] * dinv


def _tc_layer_last(h_ref, agg_ref, dinv_ref, b_ref, g_ref, bt_ref,
                   w4_ref, b4_ref, hab_ref):
    h2s = [_layer_core(h_ref, agg_ref, dinv_ref, b_ref, g_ref, bt_ref, hh)
           for hh in range(2)]
    hab_ref[...] = (jnp.dot(h2s[0], w4_ref[0], preferred_element_type=jnp.float32)
                    + jnp.dot(h2s[1], w4_ref[1],
                              preferred_element_type=jnp.float32)
                    + b4_ref[...])


def kernel(x_list, edge_index_list, edge_index_out_list, W_embed, b_embed,
           W_conv, b_conv, gamma, beta, W_fc, b_fc):
    f32 = jnp.float32
    ei = edge_index_list.astype(jnp.int32)
    eo = edge_index_out_list.astype(jnp.int32)
    E = ei.shape[1]
    L = W_conv.shape[0]

    rows = _rup(-(-E // LANES), 8 * NC * NS)
    padn = rows * LANES - E
    fill = jnp.arange(padn, dtype=jnp.int32)
    rowp = jnp.concatenate([ei[0], fill % N])
    colp = jnp.concatenate([ei[1], N + fill % (ACC_ROWS - N)])
    row3d = jnp.stack([rowp, rowp + N]).reshape(2, rows, LANES)
    col2d = colp.reshape(rows, LANES)

    orows = _rup(-(-E // LANES), 8 * NC * NS)
    opad = orows * LANES - E
    ofill = jnp.arange(opad, dtype=jnp.int32) % N
    srcp = jnp.concatenate([eo[0], ofill]).reshape(orows, LANES)
    dstp = jnp.concatenate([eo[1], ofill]).reshape(orows, LANES)

    deg_partials = _make_deg_kernel(rows)(col2d)
    degp = deg_partials.reshape(2, ACC_ROWS, 1)

    h, hs, dinv = pl.pallas_call(
        _tc_prologue,
        out_shape=(
            jax.ShapeDtypeStruct((2, N, HF), f32),
            jax.ShapeDtypeStruct((2, N, HF), f32),
            jax.ShapeDtypeStruct((N, HF), f32),
        ),
    )(x_list, W_embed, b_embed.reshape(1, DH), W_conv[0], degp)

    agg_fn = _make_agg_kernel(rows)
    for i in range(L):
        agg = agg_fn(hs.reshape(NC * N, HF), row3d, col2d).reshape(2, N, HF)
        b_i = b_conv[i].reshape(2, 1, HF)
        g_i = gamma[i].reshape(2, 1, HF)
        bt_i = beta[i].reshape(2, 1, HF)
        if i + 1 < L:
            h, hs = pl.pallas_call(
                _tc_layer,
                out_shape=(
                    jax.ShapeDtypeStruct((2, N, HF), f32),
                    jax.ShapeDtypeStruct((2, N, HF), f32),
                ),
            )(h, agg, dinv, b_i, g_i, bt_i,
              W_conv[i + 1].reshape(2, HF, DH))
        else:
            w4 = jnp.concatenate([W_fc[:DH], W_fc[DH:]], axis=1).reshape(2, HF, 4)
            b4 = jnp.concatenate([b_fc, jnp.zeros((2,), f32)]).reshape(1, 4)
            hab = pl.pallas_call(
                _tc_layer_last,
                out_shape=jax.ShapeDtypeStruct((N, 4), f32),
            )(h, agg, dinv, b_i, g_i, bt_i, w4, b4)

    eout = _make_edgeout_kernel(orows)(hab.reshape(4 * N), srcp, dstp)
    return eout[:, :E].T
